# baseline (speedup 1.0000x reference)
import functools

import jax
import jax.numpy as jnp
from jax.experimental import pallas as pl
from jax.experimental.pallas import tpu as pltpu
from jax.experimental.pallas import tpu_sc as plsc

V = 100000
D = 128
TOPK = 10
TILE = 10240
NHALF = 2
NT = 5
HPAD = NT * TILE
VPAD = NHALF * HPAD
HROWS = HPAD // D

NW = 16
L = 16
CHUNK = HPAD // NW
NSTEP = CHUNK // L
GRP = 8


def _gemv_body(wid_ref, qblk_ref, t_ref, out_ref, *, half):
    i = pl.program_id(0)
    r = wid_ref[0] % 8
    q = qblk_ref[...]
    sub = jax.lax.broadcasted_iota(jnp.int32, (8, D), 0)
    wv = jnp.sum(jnp.where(sub == r, q, 0.0), axis=0, keepdims=True)
    x = t_ref[...]
    w_rep = jnp.broadcast_to(wv.reshape(D, 1), (D, D))
    s_rep = jnp.dot(x, w_rep, preferred_element_type=jnp.float32)
    r3 = s_rep.reshape(TILE // D, D, D)
    eye = (jax.lax.broadcasted_iota(jnp.int32, (1, D, D), 1)
           == jax.lax.broadcasted_iota(jnp.int32, (1, D, D), 2)).astype(jnp.float32)
    s = jnp.sum(r3 * eye, axis=1)
    row_i = jax.lax.broadcasted_iota(jnp.int32, (TILE // D, D), 0)
    col_i = jax.lax.broadcasted_iota(jnp.int32, (TILE // D, D), 1)
    gid = half * HPAD + i * TILE + row_i * D + col_i
    out_ref[...] = jnp.where(gid < V, s, -jnp.inf)


def _gemv_half(wid, table, half):
    body = functools.partial(_gemv_body, half=half)
    return pl.pallas_call(
        body,
        grid_spec=pltpu.PrefetchScalarGridSpec(
            num_scalar_prefetch=1,
            grid=(NT,),
            in_specs=[
                pl.BlockSpec((8, D), lambda i, w: (w[0] // 8, 0)),
                pl.BlockSpec((TILE, D),
                             lambda i, w, _h=half: (_h * NT + i, 0)),
            ],
            out_specs=pl.BlockSpec((TILE // D, D), lambda i, w: (i, 0)),
        ),
        out_shape=jax.ShapeDtypeStruct((HROWS, D), jnp.float32),
    )(wid, table, table)


_SC_MESH = plsc.VectorSubcoreMesh(
    core_axis_name="c", subcore_axis_name="s", num_cores=1)
_SC_PARAMS = pltpu.CompilerParams(needs_layout_passes=False)


def _merge16(r_v, r_i, xs, xi):
    keep = (r_v > xs) | ((r_v == xs) & (r_i < xi))
    c_v = jnp.where(keep, r_v, xs)
    c_i = jnp.where(keep, r_i, xi)
    return plsc.sort_key_val(c_v, c_i, descending=False)


def _sc_local_body(scores_hbm, cand_v_out, cand_i_out, buf, stage_v, stage_i,
                   *, half):
    w = jax.lax.axis_index("s")
    base = w * CHUNK
    pltpu.sync_copy(scores_hbm.at[pl.ds(base, CHUNK)], buf)
    lane = jax.lax.iota(jnp.int32, L)
    neg = jnp.float32(-jnp.inf)
    gbase = half * HPAD + base

    def group(g, carry):
        r_v, r_i, r_min_s = carry
        xs = [buf[pl.ds((g * GRP + b) * L, L)] for b in range(GRP)]
        gmax = xs[0]
        for b in range(1, GRP):
            gmax = jnp.maximum(gmax, xs[b])
        gsort, _ = plsc.sort_key_val(gmax, lane, descending=True)

        def do_merge():
            rv, ri = r_v, r_i
            for b in range(GRP):
                ids = gbase + (g * GRP + b) * L + lane
                sx, si = plsc.sort_key_val(xs[b], ids, descending=True)
                rv, ri = _merge16(rv, ri, sx, si)
            return rv, ri, rv[0]

        return jax.lax.cond(gsort[0] > r_min_s, do_merge,
                            lambda: (r_v, r_i, r_min_s))

    init = (jnp.full((L,), neg, jnp.float32),
            jnp.zeros((L,), jnp.int32),
            neg)
    r_v, r_i, _ = jax.lax.fori_loop(0, NSTEP // GRP, group, init)

    stage_v[...] = r_v
    stage_i[...] = r_i
    pltpu.sync_copy(stage_v, cand_v_out.at[w])
    pltpu.sync_copy(stage_i, cand_i_out.at[w])


def _sc_local(half):
    return functools.partial(
        pl.kernel,
        out_type=(
            jax.ShapeDtypeStruct((NW, L), jnp.float32),
            jax.ShapeDtypeStruct((NW, L), jnp.int32),
        ),
        mesh=_SC_MESH,
        compiler_params=_SC_PARAMS,
        scratch_types=[
            pltpu.VMEM((CHUNK,), jnp.float32),
            pltpu.VMEM((L,), jnp.float32),
            pltpu.VMEM((L,), jnp.int32),
        ],
    )(functools.partial(_sc_local_body, half=half))


_sc_local_h0 = _sc_local(0)
_sc_local_h1 = _sc_local(1)


def _final_body(cv_ref, ci_ref, vals_ref, ids_ref):
    v = cv_ref[...]
    idx = ci_ref[...]
    out_col = jax.lax.broadcasted_iota(jnp.int32, (8, D), 1)
    out_row = jax.lax.broadcasted_iota(jnp.int32, (8, D), 0)

    def step(k, carry):
        v, vals, ids = carry
        m = jnp.max(v)
        am = jnp.min(jnp.where(v == m, idx, jnp.int32(2**31 - 1)))
        sel = (out_row == 0) & (out_col == k)
        vals = jnp.where(sel, m, vals)
        ids = jnp.where(sel, am, ids)
        v = jnp.where((v == m) & (idx == am), -jnp.inf, v)
        return v, vals, ids

    vals0 = jnp.full((8, D), -jnp.inf, jnp.float32)
    ids0 = jnp.zeros((8, D), jnp.int32)
    _, vals, ids = jax.lax.fori_loop(0, TOPK + 1, step, (v, vals0, ids0))
    vals_ref[...] = vals
    ids_ref[...] = ids


def kernel(wordid, table):
    wid = wordid.astype(jnp.int32)
    s0 = _gemv_half(wid, table, 0)
    s1 = _gemv_half(wid, table, 1)
    cv0, ci0 = _sc_local_h0(s0.reshape(HPAD))
    cv1, ci1 = _sc_local_h1(s1.reshape(HPAD))
    cv = jnp.concatenate([cv0, cv1], axis=0)
    ci = jnp.concatenate([ci0, ci1], axis=0)
    vals, ids = pl.pallas_call(
        _final_body,
        out_shape=(
            jax.ShapeDtypeStruct((8, D), jnp.float32),
            jax.ShapeDtypeStruct((8, D), jnp.int32),
        ),
    )(cv, ci)
    return vals[0, 1:TOPK + 1], ids[0, 1:TOPK + 1]


# speedup vs baseline: 1.0571x; 1.0571x over previous
import functools

import jax
import jax.numpy as jnp
from jax.experimental import pallas as pl
from jax.experimental.pallas import tpu as pltpu
from jax.experimental.pallas import tpu_sc as plsc

V = 100000
D = 128
TOPK = 10
TILE = 25600
NT = (V + TILE - 1) // TILE
VPAD = NT * TILE
GROWS = VPAD // D

NW = 16
L = 16
CHUNK = VPAD // NW
NSTEP = CHUNK // L
GRP = 4


def _gemv_body(wid_ref, qblk_ref, t_ref, out_ref):
    i = pl.program_id(0)
    r = wid_ref[0] % 8
    q = qblk_ref[...]
    sub = jax.lax.broadcasted_iota(jnp.int32, (8, D), 0)
    wv = jnp.sum(jnp.where(sub == r, q, 0.0), axis=0, keepdims=True)
    x = t_ref[...]
    w_rep = jnp.broadcast_to(wv.reshape(D, 1), (D, D))
    s_rep = jnp.dot(x, w_rep, preferred_element_type=jnp.float32)
    r3 = s_rep.reshape(TILE // D, D, D)
    eye = (jax.lax.broadcasted_iota(jnp.int32, (1, D, D), 1)
           == jax.lax.broadcasted_iota(jnp.int32, (1, D, D), 2)).astype(jnp.float32)
    s = jnp.sum(r3 * eye, axis=1)
    row_i = jax.lax.broadcasted_iota(jnp.int32, (TILE // D, D), 0)
    col_i = jax.lax.broadcasted_iota(jnp.int32, (TILE // D, D), 1)
    gid = i * TILE + row_i * D + col_i
    out_ref[...] = jnp.where(gid < V, s, -jnp.inf)


_SC_MESH = plsc.VectorSubcoreMesh(
    core_axis_name="c", subcore_axis_name="s", num_cores=1)
_SC_PARAMS = pltpu.CompilerParams(needs_layout_passes=False)


def _merge16(r_v, r_i, xs, xi):
    keep = (r_v > xs) | ((r_v == xs) & (r_i < xi))
    c_v = jnp.where(keep, r_v, xs)
    c_i = jnp.where(keep, r_i, xi)
    return plsc.sort_key_val(c_v, c_i, descending=False)


@functools.partial(
    pl.kernel,
    out_type=(
        jax.ShapeDtypeStruct((NW, L), jnp.float32),
        jax.ShapeDtypeStruct((NW, L), jnp.int32),
    ),
    mesh=_SC_MESH,
    compiler_params=_SC_PARAMS,
    scratch_types=[
        pltpu.VMEM((CHUNK,), jnp.float32),
        pltpu.VMEM((L,), jnp.float32),
        pltpu.VMEM((L,), jnp.int32),
    ],
)
def _sc_topk_local(scores_hbm, cand_v_out, cand_i_out, buf, stage_v, stage_i):
    w = jax.lax.axis_index("s")
    base = w * CHUNK
    pltpu.sync_copy(scores_hbm.at[pl.ds(base, CHUNK)], buf)
    lane = jax.lax.iota(jnp.int32, L)
    neg = jnp.float32(-jnp.inf)

    def group(g, carry):
        r_v, r_i, r_min_s = carry
        xs = [buf[pl.ds((g * GRP + b) * L, L)] for b in range(GRP)]
        gmax = xs[0]
        for b in range(1, GRP):
            gmax = jnp.maximum(gmax, xs[b])
        gsort, _ = plsc.sort_key_val(gmax, lane, descending=True)

        def do_merge():
            rv, ri = r_v, r_i
            for b in range(GRP):
                ids = base + (g * GRP + b) * L + lane
                sx, si = plsc.sort_key_val(xs[b], ids, descending=True)
                rv, ri = _merge16(rv, ri, sx, si)
            return rv, ri, rv[0]

        return jax.lax.cond(gsort[0] > r_min_s, do_merge,
                            lambda: (r_v, r_i, r_min_s))

    init = (jnp.full((L,), neg, jnp.float32),
            jnp.zeros((L,), jnp.int32),
            neg)
    r_v, r_i, _ = jax.lax.fori_loop(0, NSTEP // GRP, group, init)

    stage_v[...] = r_v
    stage_i[...] = r_i
    pltpu.sync_copy(stage_v, cand_v_out.at[w])
    pltpu.sync_copy(stage_i, cand_i_out.at[w])


def _final_body(cv_ref, ci_ref, vals_ref, ids_ref):
    v = cv_ref[...]
    idx = ci_ref[...]
    out_col = jax.lax.broadcasted_iota(jnp.int32, (8, D), 1)
    out_row = jax.lax.broadcasted_iota(jnp.int32, (8, D), 0)

    def step(k, carry):
        v, vals, ids = carry
        m = jnp.max(v)
        am = jnp.min(jnp.where(v == m, idx, jnp.int32(2**31 - 1)))
        sel = (out_row == 0) & (out_col == k)
        vals = jnp.where(sel, m, vals)
        ids = jnp.where(sel, am, ids)
        v = jnp.where((v == m) & (idx == am), -jnp.inf, v)
        return v, vals, ids

    vals0 = jnp.full((8, D), -jnp.inf, jnp.float32)
    ids0 = jnp.zeros((8, D), jnp.int32)
    _, vals, ids = jax.lax.fori_loop(0, TOPK + 1, step, (v, vals0, ids0))
    vals_ref[...] = vals
    ids_ref[...] = ids


def kernel(wordid, table):
    wid = wordid.astype(jnp.int32)
    scores = pl.pallas_call(
        _gemv_body,
        grid_spec=pltpu.PrefetchScalarGridSpec(
            num_scalar_prefetch=1,
            grid=(NT,),
            in_specs=[
                pl.BlockSpec((8, D), lambda i, w: (w[0] // 8, 0)),
                pl.BlockSpec((TILE, D), lambda i, w: (i, 0)),
            ],
            out_specs=pl.BlockSpec((TILE // D, D), lambda i, w: (i, 0)),
        ),
        out_shape=jax.ShapeDtypeStruct((GROWS, D), jnp.float32),
    )(wid, table, table)

    cv, ci = _sc_topk_local(scores.reshape(VPAD))
    vals, ids = pl.pallas_call(
        _final_body,
        out_shape=(
            jax.ShapeDtypeStruct((8, D), jnp.float32),
            jax.ShapeDtypeStruct((8, D), jnp.int32),
        ),
    )(cv, ci)
    return vals[0, 1:TOPK + 1], ids[0, 1:TOPK + 1]


# speedup vs baseline: 1.0939x; 1.0347x over previous
import functools

import jax
import jax.numpy as jnp
from jax.experimental import pallas as pl
from jax.experimental.pallas import tpu as pltpu
from jax.experimental.pallas import tpu_sc as plsc

V = 100000
D = 128
TOPK = 10
TILE = 25600
NT = (V + TILE - 1) // TILE
VPAD = NT * TILE
GROWS = VPAD // D

NC = 2
NW = 16 * NC
L = 16
CHUNK = VPAD // NW
NSTEP = CHUNK // L
GRP = 4


def _gemv_body(wid_ref, qblk_ref, t_ref, out_ref):
    i = pl.program_id(0)
    r = wid_ref[0] % 8
    q = qblk_ref[...]
    sub = jax.lax.broadcasted_iota(jnp.int32, (8, D), 0)
    wv = jnp.sum(jnp.where(sub == r, q, 0.0), axis=0, keepdims=True)
    x = t_ref[...]
    w_rep = jnp.broadcast_to(wv.reshape(D, 1), (D, D))
    s_rep = jnp.dot(x, w_rep, preferred_element_type=jnp.float32)
    r3 = s_rep.reshape(TILE // D, D, D)
    eye = (jax.lax.broadcasted_iota(jnp.int32, (1, D, D), 1)
           == jax.lax.broadcasted_iota(jnp.int32, (1, D, D), 2)).astype(jnp.float32)
    s = jnp.sum(r3 * eye, axis=1)
    row_i = jax.lax.broadcasted_iota(jnp.int32, (TILE // D, D), 0)
    col_i = jax.lax.broadcasted_iota(jnp.int32, (TILE // D, D), 1)
    gid = i * TILE + row_i * D + col_i
    out_ref[...] = jnp.where(gid < V, s, -jnp.inf)


_SC_MESH = plsc.VectorSubcoreMesh(
    core_axis_name="c", subcore_axis_name="s", num_cores=NC)
_SC_PARAMS = pltpu.CompilerParams(needs_layout_passes=False)


def _merge16(r_v, r_i, xs, xi):
    keep = (r_v > xs) | ((r_v == xs) & (r_i < xi))
    c_v = jnp.where(keep, r_v, xs)
    c_i = jnp.where(keep, r_i, xi)
    return plsc.sort_key_val(c_v, c_i, descending=False)


@functools.partial(
    pl.kernel,
    out_type=(
        jax.ShapeDtypeStruct((NW, L), jnp.float32),
        jax.ShapeDtypeStruct((NW, L), jnp.int32),
    ),
    mesh=_SC_MESH,
    compiler_params=_SC_PARAMS,
    scratch_types=[
        pltpu.VMEM((CHUNK,), jnp.float32),
        pltpu.VMEM((L,), jnp.float32),
        pltpu.VMEM((L,), jnp.int32),
    ],
)
def _sc_topk_local(scores_hbm, cand_v_out, cand_i_out, buf, stage_v, stage_i):
    w = jax.lax.axis_index("s") * NC + jax.lax.axis_index("c")
    base = w * CHUNK
    pltpu.sync_copy(scores_hbm.at[pl.ds(base, CHUNK)], buf)
    lane = jax.lax.iota(jnp.int32, L)
    neg = jnp.float32(-jnp.inf)

    def group(g, carry):
        r_v, r_i, r_min_s = carry
        xs = [buf[pl.ds((g * GRP + b) * L, L)] for b in range(GRP)]
        gmax = xs[0]
        for b in range(1, GRP):
            gmax = jnp.maximum(gmax, xs[b])
        gsort, _ = plsc.sort_key_val(gmax, lane, descending=True)

        def do_merge():
            rv, ri = r_v, r_i
            for b in range(GRP):
                ids = base + (g * GRP + b) * L + lane
                sx, si = plsc.sort_key_val(xs[b], ids, descending=True)
                rv, ri = _merge16(rv, ri, sx, si)
            return rv, ri, rv[0]

        return jax.lax.cond(gsort[0] > r_min_s, do_merge,
                            lambda: (r_v, r_i, r_min_s))

    init = (jnp.full((L,), neg, jnp.float32),
            jnp.zeros((L,), jnp.int32),
            neg)
    r_v, r_i, _ = jax.lax.fori_loop(0, NSTEP // GRP, group, init)

    stage_v[...] = r_v
    stage_i[...] = r_i
    pltpu.sync_copy(stage_v, cand_v_out.at[w])
    pltpu.sync_copy(stage_i, cand_i_out.at[w])


def _final_body(cv_ref, ci_ref, vals_ref, ids_ref):
    v = cv_ref[...]
    idx = ci_ref[...]
    out_col = jax.lax.broadcasted_iota(jnp.int32, (8, D), 1)
    out_row = jax.lax.broadcasted_iota(jnp.int32, (8, D), 0)

    def step(k, carry):
        v, vals, ids = carry
        m = jnp.max(v)
        am = jnp.min(jnp.where(v == m, idx, jnp.int32(2**31 - 1)))
        sel = (out_row == 0) & (out_col == k)
        vals = jnp.where(sel, m, vals)
        ids = jnp.where(sel, am, ids)
        v = jnp.where((v == m) & (idx == am), -jnp.inf, v)
        return v, vals, ids

    vals0 = jnp.full((8, D), -jnp.inf, jnp.float32)
    ids0 = jnp.zeros((8, D), jnp.int32)
    _, vals, ids = jax.lax.fori_loop(0, TOPK + 1, step, (v, vals0, ids0))
    vals_ref[...] = vals
    ids_ref[...] = ids


def kernel(wordid, table):
    wid = wordid.astype(jnp.int32)
    scores = pl.pallas_call(
        _gemv_body,
        grid_spec=pltpu.PrefetchScalarGridSpec(
            num_scalar_prefetch=1,
            grid=(NT,),
            in_specs=[
                pl.BlockSpec((8, D), lambda i, w: (w[0] // 8, 0)),
                pl.BlockSpec((TILE, D), lambda i, w: (i, 0)),
            ],
            out_specs=pl.BlockSpec((TILE // D, D), lambda i, w: (i, 0)),
        ),
        out_shape=jax.ShapeDtypeStruct((GROWS, D), jnp.float32),
    )(wid, table, table)

    cv, ci = _sc_topk_local(scores.reshape(VPAD))
    vals, ids = pl.pallas_call(
        _final_body,
        out_shape=(
            jax.ShapeDtypeStruct((8, D), jnp.float32),
            jax.ShapeDtypeStruct((8, D), jnp.int32),
        ),
    )(cv, ci)
    return vals[0, 1:TOPK + 1], ids[0, 1:TOPK + 1]
